# Initial kernel scaffold; baseline (speedup 1.0000x reference)
import jax
import jax.numpy as jnp
from jax.experimental import pallas as pl


def kernel(user_num, item_num, user_cat, item_cat, history_items, Wun, bun, ut0, ut1, ut2, Wim, bim, it0, it1, it2, hist_tab, Wattn, W1, b1, W2, b2, W3, b3):
    raise NotImplementedError("write your pallas kernel here")


# speedup vs baseline: 1.9404x; 1.9404x over previous
import functools

import jax
import jax.numpy as jnp
from jax import lax
from jax.experimental import pallas as pl
from jax.experimental.pallas import tpu as pltpu
from jax.experimental.pallas import tpu_sc as plsc

B = 4096
D = 64
L = 50
NU = 16
NI = 16
V = 100000
H1 = 512
H2 = 256

NC = 2
NS = 16
NW = NC * NS
CB = B // NW
SW = 128
NSTREAM = (B * L) // (NW * SW)


def _sc_gather_body(hidx, cidx, ut0, ut1, ut2, it0, it1, it2, htab,
                    hist_out, cat_out, idx_v, cidx_v, rows_v, crow_v, sem):
    c = lax.axis_index("c")
    s = lax.axis_index("s")
    w = s * NC + c
    pltpu.sync_copy(hidx.at[w], idx_v)
    pltpu.sync_copy(cidx.at[w], cidx_v)
    tabs = (ut0, ut1, ut2, it0, it1, it2)
    for t in range(6):
        pltpu.async_copy(tabs[t].at[cidx_v.at[t]], crow_v, sem).wait()
        pltpu.sync_copy(crow_v, cat_out.at[t, pl.ds(w * CB, CB)])

    def step(j, carry):
        pltpu.async_copy(htab.at[idx_v.at[j]], rows_v, sem).wait()
        pltpu.sync_copy(rows_v,
                        hist_out.at[pl.ds((w * NSTREAM + j) * SW, SW)])
        return carry

    lax.fori_loop(0, NSTREAM, step, 0)


@functools.partial(jax.jit, static_argnames=())
def _sc_gather(hidx, cidx, ut0, ut1, ut2, it0, it1, it2, htab):
    mesh = plsc.VectorSubcoreMesh(core_axis_name="c", subcore_axis_name="s")
    f = functools.partial(
        pl.kernel,
        out_type=(
            jax.ShapeDtypeStruct((B * L, D), jnp.float32),
            jax.ShapeDtypeStruct((6, B, D), jnp.float32),
        ),
        mesh=mesh,
        scratch_types=[
            pltpu.VMEM((NSTREAM, SW), jnp.int32),
            pltpu.VMEM((6, CB), jnp.int32),
            pltpu.VMEM((SW, D), jnp.float32),
            pltpu.VMEM((CB, D), jnp.float32),
            pltpu.SemaphoreType.DMA,
        ],
        compiler_params=pltpu.CompilerParams(use_tc_tiling_on_sc=False),
    )(_sc_gather_body)
    return f(hidx, cidx, ut0, ut1, ut2, it0, it1, it2, htab)


R = 256


def _tc_body(un_ref, inum_ref, cat_ref, hist_ref,
             Wun_ref, bun_ref, Wim_ref, bim_ref, wattn_ref,
             W1_ref, b1_ref, W2_ref, b2_ref, W3_ref, b3_ref, out_ref):
    f32 = jnp.float32
    ue = (jnp.dot(un_ref[...], Wun_ref[...], preferred_element_type=f32)
          + bun_ref[...] + cat_ref[0] + cat_ref[1] + cat_ref[2])
    ie = (jnp.dot(inum_ref[...], Wim_ref[...], preferred_element_type=f32)
          + bim_ref[...] + cat_ref[3] + cat_ref[4] + cat_ref[5])
    hist = hist_ref[...]
    qw = ie * wattn_ref[...]
    scores = jnp.sum(hist * qw[:, None, :], axis=2)
    e = jnp.exp(jnp.tanh(scores))
    wts = e / jnp.sum(e, axis=1, keepdims=True)
    att = jnp.sum(wts[:, :, None] * hist, axis=1)
    comb = jnp.concatenate([ue, ie, att], axis=1)
    h = jnp.maximum(jnp.dot(comb, W1_ref[...], preferred_element_type=f32)
                    + b1_ref[...], 0.0)
    h = jnp.maximum(jnp.dot(h, W2_ref[...], preferred_element_type=f32)
                    + b2_ref[...], 0.0)
    logits = jnp.dot(h, W3_ref[...], preferred_element_type=f32) + b3_ref[...]
    out_ref[...] = jax.nn.sigmoid(logits)


def _tc_fused(user_num, item_num, cat_emb, hist_emb,
              Wun, bun, Wim, bim, wattn, W1, b1, W2, b2, W3, b3):
    grid = (B // R,)
    full = lambda shape: pl.BlockSpec(shape, lambda i: (0,) * len(shape))
    return pl.pallas_call(
        _tc_body,
        grid=grid,
        in_specs=[
            pl.BlockSpec((R, NU), lambda i: (i, 0)),
            pl.BlockSpec((R, NI), lambda i: (i, 0)),
            pl.BlockSpec((6, R, D), lambda i: (0, i, 0)),
            pl.BlockSpec((R, L, D), lambda i: (i, 0, 0)),
            full((NU, D)), full((1, D)),
            full((NI, D)), full((1, D)), full((1, D)),
            full((3 * D, H1)), full((1, H1)),
            full((H1, H2)), full((1, H2)),
            full((H2, 1)), full((1, 1)),
        ],
        out_specs=pl.BlockSpec((R, 1), lambda i: (i, 0)),
        out_shape=jax.ShapeDtypeStruct((B, 1), jnp.float32),
    )(user_num, item_num, cat_emb, hist_emb,
      Wun, bun, Wim, bim, wattn, W1, b1, W2, b2, W3, b3)


def kernel(user_num, item_num, user_cat, item_cat, history_items,
           Wun, bun, ut0, ut1, ut2, Wim, bim, it0, it1, it2,
           hist_tab, Wattn, W1, b1, W2, b2, W3, b3):
    hidx = history_items.astype(jnp.int32).reshape(NW, NSTREAM, SW)
    catT = jnp.concatenate([user_cat.T, item_cat.T], axis=0).astype(jnp.int32)
    cidx = catT.reshape(6, NW, CB).transpose(1, 0, 2)
    hist_emb, cat_emb = _sc_gather(hidx, cidx, ut0, ut1, ut2,
                                   it0, it1, it2, hist_tab)
    out = _tc_fused(user_num, item_num, cat_emb, hist_emb.reshape(B, L, D),
                    Wun, bun.reshape(1, D), Wim, bim.reshape(1, D),
                    Wattn.reshape(1, D), W1, b1.reshape(1, H1),
                    W2, b2.reshape(1, H2), W3, b3.reshape(1, 1))
    return out.reshape(B)
